# Initial kernel scaffold; baseline (speedup 1.0000x reference)
import jax
import jax.numpy as jnp
from jax.experimental import pallas as pl


def kernel(user_emb, item_emb, adj_indices, adj_values):
    raise NotImplementedError("write your pallas kernel here")


# speedup vs baseline: 2.9879x; 2.9879x over previous
import functools

import jax
import jax.numpy as jnp
from jax import lax
from jax.experimental import pallas as pl
from jax.experimental.pallas import tpu as pltpu
from jax.experimental.pallas import tpu_sc as plsc

USER_N = 15000
ITEM_N = 35000
N_NODES = USER_N + ITEM_N
D = 64
E = 800000

HALF = N_NODES // 2
PAD = 88
HALF_P = HALF + PAD
N_P = 2 * HALF_P

NSUB = 16
RPT = HALF_P // NSUB
WCH = 49
NWCH = RPT // WCH

ECH = 128
EPT = 50176
E_P = NSUB * EPT
NECH = EPT // ECH

_mesh = plsc.VectorSubcoreMesh(core_axis_name="c", subcore_axis_name="s")


def _layer_body(do_mean, ego, rows, cols, vals, e1, e2, out,
                acc, cidx, ridx, evals, gath, ba, bb, bc, sem):
    cid = lax.axis_index("c")
    sid = lax.axis_index("s")
    base = cid * HALF

    z16 = jnp.zeros((16,), jnp.float32)

    def zrow(r, carry):
        for c in range(4):
            ba[r, pl.ds(c * 16, 16)] = z16
        return carry

    lax.fori_loop(0, WCH, zrow, 0)
    for j in range(NWCH):
        pltpu.sync_copy(ba, acc.at[pl.ds(sid * RPT + j * WCH, WCH)])
    plsc.subcore_barrier()

    ebase = sid * EPT

    def chunk(k, carry):
        off = ebase + k * ECH
        pltpu.sync_copy(rows.at[pl.ds(off, ECH)], ridx)
        pltpu.sync_copy(cols.at[pl.ds(off, ECH)], cidx)
        pltpu.sync_copy(vals.at[pl.ds(off, ECH)], evals)
        for g in range(ECH // 16):
            sl = pl.ds(g * 16, 16)
            c16 = cidx[sl]
            cidx[sl] = jnp.where(c16 >= HALF, c16 + PAD, c16)
        pltpu.async_copy(ego.at[cidx], gath, sem).wait()
        for g in range(ECH // 16):
            sl = pl.ds(g * 16, 16)
            r16 = ridx[sl]
            loc = r16 - base
            ok = (loc >= 0) & (loc < HALF)
            loc = jnp.where(loc < 0, loc + HALF, loc)
            loc = jnp.where(loc >= HALF, loc - HALF, loc)
            ridx[sl] = loc
            v16 = evals[sl]
            evals[sl] = jnp.where(ok, v16, jnp.float32(0.0))
        dnums = lax.GatherDimensionNumbers(
            offset_dims=(), collapsed_slice_dims=(0,), start_index_map=(0,))
        for g in range(ECH // 16):
            v16 = evals[pl.ds(g * 16, 16)]
            for j in range(16):
                vb = lax.gather(
                    v16, jnp.full((16, 1), j, jnp.int32), dnums, (1,),
                    mode=lax.GatherScatterMode.PROMISE_IN_BOUNDS)
                r = g * 16 + j
                for c in range(4):
                    sl = pl.ds(c * 16, 16)
                    gath[r, sl] = gath[r, sl] * vb
        pltpu.sync_copy(gath, acc.at[ridx], add=True)
        return carry

    lax.fori_loop(0, NECH, chunk, 0)
    plsc.subcore_barrier()

    obase = cid * HALF_P + sid * RPT
    if not do_mean:
        pltpu.sync_copy(acc.at[pl.ds(sid * RPT, RPT)],
                        out.at[pl.ds(obase, RPT)])
    else:
        inv3 = jnp.float32(1.0 / 3.0)
        for j in range(NWCH):
            a0 = sid * RPT + j * WCH
            g0 = obase + j * WCH
            pltpu.sync_copy(acc.at[pl.ds(a0, WCH)], ba)
            pltpu.sync_copy(e1.at[pl.ds(g0, WCH)], bb)
            pltpu.sync_copy(e2.at[pl.ds(g0, WCH)], bc)

            def mrow(r, carry):
                for c in range(4):
                    sl = pl.ds(c * 16, 16)
                    ba[r, sl] = (ba[r, sl] + bb[r, sl] + bc[r, sl]) * inv3
                return carry

            lax.fori_loop(0, WCH, mrow, 0)
            pltpu.sync_copy(ba, out.at[pl.ds(g0, WCH)])


def _make_layer(do_mean):
    return pl.kernel(
        functools.partial(_layer_body, do_mean),
        out_type=jax.ShapeDtypeStruct((N_P, D), jnp.float32),
        mesh=_mesh,
        compiler_params=pltpu.CompilerParams(use_tc_tiling_on_sc=False),
        scratch_types=[
            pltpu.VMEM_SHARED((HALF_P, D), jnp.float32),
            pltpu.VMEM((ECH,), jnp.int32),
            pltpu.VMEM((ECH,), jnp.int32),
            pltpu.VMEM((ECH,), jnp.float32),
            pltpu.VMEM((ECH, D), jnp.float32),
            pltpu.VMEM((WCH, D), jnp.float32),
            pltpu.VMEM((WCH, D), jnp.float32),
            pltpu.VMEM((WCH, D), jnp.float32),
            pltpu.SemaphoreType.DMA,
        ],
    )


_layer = _make_layer(False)
_layer_mean = _make_layer(True)


@jax.jit
def _run(user_emb, item_emb, adj_indices, adj_values):
    rows = adj_indices[0].astype(jnp.int32)
    cols = adj_indices[1].astype(jnp.int32)
    vals = adj_values.astype(jnp.float32)
    epad = E_P - E
    rows_p = jnp.concatenate([rows, jnp.zeros((epad,), jnp.int32)])
    cols_p = jnp.concatenate([cols, jnp.zeros((epad,), jnp.int32)])
    vals_p = jnp.concatenate([vals, jnp.zeros((epad,), jnp.float32)])
    zpad = jnp.zeros((PAD, D), jnp.float32)
    ego0 = jnp.concatenate(
        [user_emb, item_emb[:HALF - USER_N], zpad,
         item_emb[HALF - USER_N:], zpad], axis=0)
    e1 = _layer(ego0, rows_p, cols_p, vals_p, ego0, ego0)
    e2 = _layer(e1, rows_p, cols_p, vals_p, ego0, ego0)
    e3m = _layer_mean(e2, rows_p, cols_p, vals_p, e1, e2)
    user_all = e3m[:USER_N]
    item_all = jnp.concatenate(
        [e3m[USER_N:HALF], e3m[HALF_P:HALF_P + HALF]], axis=0)
    return user_all, item_all


def kernel(user_emb, item_emb, adj_indices, adj_values):
    return _run(user_emb, item_emb, adj_indices, adj_values)
